# baseline (speedup 1.0000x reference)
import jax
import jax.numpy as jnp
from jax import lax
from jax.experimental import pallas as pl
from jax.experimental.pallas import tpu as pltpu
from jax.experimental.pallas import tpu_sc as plsc

LM = 128
W = 2 * LM - 1
PACKED = LM * LM
BATCH = 1024
NW = 32
ROWS = BATCH // NW

IN_OFF = 128
IN_PAD = PACKED + IN_OFF + 16
OUT_ROWS = LM + 1


def _vec(inA, inB, outA, outB, lane, j):
    l = j >> 4
    col = (j & 15) * 16
    src0 = l * l + l - 127 + IN_OFF + col
    dv = lane + (col - 127)
    m = jnp.abs(dv) <= l
    vA = inA[pl.ds(src0, 16)]
    vB = inB[pl.ds(src0, 16)]
    outA[l, pl.ds(col, 16)] = jnp.where(m, vA, 0.0)
    outB[l, pl.ds(col, 16)] = jnp.where(m, vB, 0.0)


def _copy_pair(inA, inB, outA, outB, lane):
    @plsc.parallel_loop(0, 16 * (LM - 1), unroll=8)
    def _(j):
        _vec(inA, inB, outA, outB, lane, j)

    def last(j, c):
        _vec(inA, inB, outA, outB, lane, j)
        return c

    lax.fori_loop(16 * (LM - 1), 16 * LM, last, 0)


def _make_body(row0, rows_per_tile):
    def _body(in_hbm, out_hbm, in0, in1, out0, out1, si0, si1, so0, so1):
        wid = lax.axis_index("s") * 2 + lax.axis_index("c")
        base = wid * rows_per_tile
        ins = (in0, in1)
        outs = (out0, out1)
        sis = (si0, si1)
        sos = (so0, so1)
        lane = lax.iota(jnp.int32, 16)

        def start_in(p, b):
            pltpu.make_async_copy(
                in_hbm.at[pl.ds((row0 + base + b) * PACKED, PACKED)],
                ins[p].at[pl.ds(IN_OFF, PACKED)], sis[p]
            ).start()

        def wait_in(p, b):
            pltpu.make_async_copy(
                in_hbm.at[pl.ds((row0 + base + b) * PACKED, PACKED)],
                ins[p].at[pl.ds(IN_OFF, PACKED)], sis[p]
            ).wait()

        def start_out(p, b):
            pltpu.make_async_copy(
                outs[p].at[pl.ds(0, LM)], out_hbm.at[base + b], sos[p]
            ).start()

        def wait_out(p, b):
            pltpu.make_async_copy(
                outs[p].at[pl.ds(0, LM)], out_hbm.at[base + b], sos[p]
            ).wait()

        for p in range(2):
            start_in(p, p)

        def pair(i, c):
            for p in range(2):
                wait_in(p, 2 * i + p)

            @pl.when(i > 0)
            def _():
                for p in range(2):
                    wait_out(p, 2 * i - 2 + p)

            _copy_pair(in0, in1, out0, out1, lane)

            for p in range(2):
                start_out(p, 2 * i + p)

            @pl.when(2 * i + 2 < rows_per_tile)
            def _():
                for p in range(2):
                    start_in(p, 2 * i + 2 + p)
            return c

        lax.fori_loop(0, rows_per_tile // 2, pair, 0)

        for p in range(2):
            wait_out(p, rows_per_tile - 2 + p)

    return _body


def _make_call(row0, nrows):
    mesh = plsc.VectorSubcoreMesh(core_axis_name="c", subcore_axis_name="s")
    return pl.kernel(
        _make_body(row0, nrows // NW),
        mesh=mesh,
        out_type=jax.ShapeDtypeStruct((nrows, LM, W), jnp.float32),
        scratch_types=[
            pltpu.VMEM((IN_PAD,), jnp.float32),
            pltpu.VMEM((IN_PAD,), jnp.float32),
            pltpu.VMEM((OUT_ROWS, W), jnp.float32),
            pltpu.VMEM((OUT_ROWS, W), jnp.float32),
            pltpu.SemaphoreType.DMA,
            pltpu.SemaphoreType.DMA,
            pltpu.SemaphoreType.DMA,
            pltpu.SemaphoreType.DMA,
        ],
    )


NSPLIT = 2


def kernel(tensor):
    flat = tensor.reshape(BATCH * PACKED)
    chunk = BATCH // NSPLIT
    parts = [_make_call(s * chunk, chunk)(flat) for s in range(NSPLIT)]
    return jnp.concatenate(parts, axis=0)


# speedup vs baseline: 1.5354x; 1.5354x over previous
import jax
import jax.numpy as jnp
from jax import lax
from jax.experimental import pallas as pl
from jax.experimental.pallas import tpu as pltpu
from jax.experimental.pallas import tpu_sc as plsc

LM = 128
W = 2 * LM - 1
PACKED = LM * LM
BATCH = 1024
NW = 32
ROWS = BATCH // NW

IN_OFF = 128
IN_PAD = PACKED + IN_OFF + 16
OUT_ROWS = LM + 1


def _vec(inA, inB, outA, outB, lane, j):
    l = j >> 4
    col = (j & 15) * 16
    src0 = l * l + l - 127 + IN_OFF + col
    dv = lane + (col - 127)
    m = jnp.abs(dv) <= l
    vA = inA[pl.ds(src0, 16)]
    vB = inB[pl.ds(src0, 16)]
    outA[l, pl.ds(col, 16)] = jnp.where(m, vA, 0.0)
    outB[l, pl.ds(col, 16)] = jnp.where(m, vB, 0.0)


def _copy_pair(inA, inB, outA, outB, lane):
    @plsc.parallel_loop(0, 16 * (LM - 1), unroll=8)
    def _(j):
        _vec(inA, inB, outA, outB, lane, j)

    def last(j, c):
        _vec(inA, inB, outA, outB, lane, j)
        return c

    lax.fori_loop(16 * (LM - 1), 16 * LM, last, 0)


def _body(in_hbm, out_hbm, in0, in1, out0, out1, si0, si1, so0, so1):
    wid = lax.axis_index("s") * 2 + lax.axis_index("c")
    base = wid * ROWS
    ins = (in0, in1)
    outs = (out0, out1)
    sis = (si0, si1)
    sos = (so0, so1)
    lane = lax.iota(jnp.int32, 16)

    def start_in(p, b):
        hb = (b >> 3) * 131072 + (b & 7) * 128

        def chunk(cc, c2):
            pltpu.make_async_copy(
                in_hbm.at[pl.ds(hb + cc * 1024, 128)],
                ins[p].at[pl.ds(IN_OFF + cc * 128, 128)], sis[p]
            ).start()
            return c2

        lax.fori_loop(0, 128, chunk, 0, unroll=8)

    def wait_in(p):
        pltpu.make_async_copy(
            in_hbm.at[pl.ds(0, PACKED)],
            ins[p].at[pl.ds(IN_OFF, PACKED)], sis[p]
        ).wait()

    for p in range(2):
        start_in(p, base + p)

    def pair(i, c):
        b = base + 2 * i
        for p in range(2):
            wait_in(p)

        @pl.when(i > 0)
        def _():
            for p in range(2):
                pltpu.make_async_copy(
                    outs[p].at[pl.ds(0, LM)], out_hbm.at[b - 2 + p], sos[p]
                ).wait()

        _copy_pair(in0, in1, out0, out1, lane)

        for p in range(2):
            pltpu.make_async_copy(
                outs[p].at[pl.ds(0, LM)], out_hbm.at[b + p], sos[p]
            ).start()

        @pl.when(2 * i + 2 < ROWS)
        def _():
            for p in range(2):
                start_in(p, b + 2 + p)
        return c

    lax.fori_loop(0, ROWS // 2, pair, 0)

    for p in range(2):
        pltpu.make_async_copy(
            outs[p].at[pl.ds(0, LM)], out_hbm.at[base + ROWS - 2 + p], sos[p]
        ).wait()


def kernel(tensor):
    mesh = plsc.VectorSubcoreMesh(core_axis_name="c", subcore_axis_name="s")
    k = pl.kernel(
        _body,
        mesh=mesh,
        out_type=jax.ShapeDtypeStruct((BATCH, LM, W), jnp.float32),
        scratch_types=[
            pltpu.VMEM((IN_PAD,), jnp.float32),
            pltpu.VMEM((IN_PAD,), jnp.float32),
            pltpu.VMEM((OUT_ROWS, W), jnp.float32),
            pltpu.VMEM((OUT_ROWS, W), jnp.float32),
            pltpu.SemaphoreType.DMA,
            pltpu.SemaphoreType.DMA,
            pltpu.SemaphoreType.DMA,
            pltpu.SemaphoreType.DMA,
        ],
    )
    flat = tensor.reshape(128, 8, 128, 128).transpose(0, 2, 1, 3)
    return k(flat.reshape(BATCH * PACKED))
